# baseline (speedup 1.0000x reference)
import dataclasses

import jax
import jax.numpy as jnp
from jax.experimental import pallas as pl
from jax.experimental.pallas import tpu as pltpu
from jax.experimental.pallas import tpu_sc as plsc

_SC_PARAMS = pltpu.CompilerParams()
if "needs_layout_passes" in pltpu.CompilerParams.__dataclass_fields__:
    _SC_PARAMS = dataclasses.replace(_SC_PARAMS, needs_layout_passes=False)


_VP = 104


def _tables_body(z_ref, emb_ref, wt1_ref, wt2_ref, wt3_ref, wrbft_ref,
                 brbf_ref, bdense_ref, h_ref, ab_ref, c_ref):
    n = z_ref.shape[0]
    v = emb_ref.shape[0]
    emb = emb_ref[...]
    if v < 128:
        emb = jnp.concatenate(
            [emb, jnp.zeros((128 - v, emb.shape[1]), emb.dtype)], axis=0)
    lane = jax.lax.broadcasted_iota(jnp.int32, (n, 128), 1)
    oh = (z_ref[...] == lane).astype(jnp.float32)
    h_ref[...] = jnp.dot(oh, emb, preferred_element_type=jnp.float32)
    bias = jnp.dot(brbf_ref[...], wt3_ref[...],
                   preferred_element_type=jnp.float32) + bdense_ref[...]
    ab_ref[:_VP, :] = (jnp.dot(emb, wt1_ref[...],
                               preferred_element_type=jnp.float32)
                       + bias)[:_VP].astype(jnp.float8_e4m3fn)
    ab_ref[_VP:, :] = jnp.dot(emb, wt2_ref[...],
                              preferred_element_type=jnp.float32
                              )[:_VP].astype(jnp.float8_e4m3fn)
    c_ref[...] = jnp.dot(wrbft_ref[...], wt3_ref[...],
                         preferred_element_type=jnp.float32
                         ).astype(jnp.bfloat16)


def _edge_body(zs_ref, zd_ref, rbft_ref, ab_ref, c_ref, *rest):
    m_ref = rest[-1]
    be = m_ref.shape[0]
    sub = jax.lax.broadcasted_iota(jnp.int32, (_VP, be), 0)
    zs = jnp.broadcast_to(zs_ref[...].reshape(1, be), (_VP, be))
    zd = jnp.broadcast_to(zd_ref[...].reshape(1, be), (_VP, be))
    oht = jnp.concatenate(
        [(zs == sub), (zd == sub)], axis=0).astype(jnp.float8_e4m3fn)
    m_ref[...] = (jax.lax.dot_general(
        oht, ab_ref[...], (((0,), (0,)), ((), ())),
        preferred_element_type=jnp.float32)
        + jax.lax.dot_general(
            rbft_ref[...], c_ref[...], (((0,), (0,)), ((), ())),
            preferred_element_type=jnp.float32))


def kernel(Z, edge_index, rbf, emb_table, W_rbf, b_rbf, W_dense, b_dense):
    N = Z.shape[0]
    E = edge_index.shape[1]
    EMB = emb_table.shape[1]
    NR = rbf.shape[1]

    Wd_t = W_dense.T
    Wt1 = Wd_t[:EMB]
    Wt2 = Wd_t[EMB:2 * EMB]
    Wt3 = Wd_t[2 * EMB:]
    Wrbf_t = W_rbf.T
    Zi = Z.astype(jnp.int32)
    Zc = Zi.reshape(N, 1)

    h, ab, C = pl.pallas_call(
        _tables_body,
        out_shape=(
            jax.ShapeDtypeStruct((N, EMB), jnp.float32),
            jax.ShapeDtypeStruct((2 * _VP, EMB), jnp.float8_e4m3fn),
            jax.ShapeDtypeStruct((NR, EMB), jnp.bfloat16),
        ),
    )(Zc, emb_table, Wt1, Wt2, Wt3, Wrbf_t,
      b_rbf.reshape(1, EMB), b_dense.reshape(1, EMB))

    rbft = rbf.T.astype(jnp.bfloat16)


    mesh = plsc.VectorSubcoreMesh(core_axis_name="c", subcore_axis_name="s")
    W = 512
    NCH = 2
    EC = E // NCH

    @pl.kernel(out_type=jax.ShapeDtypeStruct((2 * EC // W, W), jnp.int32),
               mesh=mesh,
               compiler_params=_SC_PARAMS,
               scratch_types=[pltpu.VMEM((N,), jnp.int32),
                              pltpu.SemaphoreType.DMA])
    def _classids(z_hbm, i_hbm, o_hbm, z_vmem, sem):
        pltpu.async_copy(z_hbm, z_vmem, sem).wait()

        def body(i_vmem, o_vmem):
            irow = i_vmem.at[0]
            orow = o_vmem.at[0]

            @pl.loop(0, W, step=16)
            def _(k):
                vidx = irow[pl.ds(k, 16)]
                orow[pl.ds(k, 16)] = plsc.load_gather(z_vmem, [vidx])

        pltpu.emit_pipeline(
            body,
            grid=(2 * EC // W,),
            in_specs=[pl.BlockSpec((1, W), lambda i: (0, i))],
            out_specs=[pl.BlockSpec((1, W), lambda i: (i, 0))],
            core_axis_name=("c", "s"),
            dimension_semantics=(pltpu.PARALLEL,),
        )(i_hbm, o_hbm)

    zsd_chunks = [
        _classids(Zi, edge_index[:, k * EC:(k + 1) * EC]
                  .reshape(1, 2 * EC).astype(jnp.int32)).reshape(2, 1, EC)
        for k in range(NCH)
    ]

    BE = 16000
    NBC = EC // BE
    m = None
    for k in range(NCH):
        kk = k
        args = [zsd_chunks[k], zsd_chunks[k], rbft, ab, C]
        in_specs = [
            pl.BlockSpec((1, 1, BE), lambda i: (0, 0, i)),
            pl.BlockSpec((1, 1, BE), lambda i: (1, 0, i)),
            pl.BlockSpec((NR, BE), lambda i, kk=kk: (0, i + kk * NBC)),
            pl.BlockSpec((2 * _VP, EMB), lambda i: (0, 0)),
            pl.BlockSpec((NR, EMB), lambda i: (0, 0)),
        ]
        aliases = {}
        if m is not None:
            args.append(m)
            in_specs.append(pl.BlockSpec(memory_space=pltpu.MemorySpace.HBM))
            aliases = {5: 0}
        m = pl.pallas_call(
            _edge_body,
            grid=(NBC,),
            in_specs=in_specs,
            out_specs=pl.BlockSpec((BE, EMB), lambda i, kk=kk: (i + kk * NBC, 0)),
            out_shape=jax.ShapeDtypeStruct((E, EMB), jnp.float32),
            input_output_aliases=aliases,
        )(*args)

    return (h, m)


# speedup vs baseline: 1.1016x; 1.1016x over previous
import dataclasses

import jax
import jax.numpy as jnp
from jax.experimental import pallas as pl
from jax.experimental.pallas import tpu as pltpu
from jax.experimental.pallas import tpu_sc as plsc

_SC_PARAMS = pltpu.CompilerParams()
if "needs_layout_passes" in pltpu.CompilerParams.__dataclass_fields__:
    _SC_PARAMS = dataclasses.replace(_SC_PARAMS, needs_layout_passes=False)


_VP = 104


def _tables_body(z_ref, emb_ref, wt1_ref, wt2_ref, wt3_ref, wrbft_ref,
                 brbf_ref, bdense_ref, h_ref, ab_ref, c_ref):
    n = z_ref.shape[0]
    v = emb_ref.shape[0]
    emb = emb_ref[...]
    if v < 128:
        emb = jnp.concatenate(
            [emb, jnp.zeros((128 - v, emb.shape[1]), emb.dtype)], axis=0)
    lane = jax.lax.broadcasted_iota(jnp.int32, (n, 128), 1)
    oh = (z_ref[...] == lane).astype(jnp.float32)
    h_ref[...] = jnp.dot(oh, emb, preferred_element_type=jnp.float32)
    bias = jnp.dot(brbf_ref[...], wt3_ref[...],
                   preferred_element_type=jnp.float32) + bdense_ref[...]
    ab_ref[:_VP, :] = (jnp.dot(emb, wt1_ref[...],
                               preferred_element_type=jnp.float32)
                       + bias)[:_VP].astype(jnp.float8_e4m3fn)
    ab_ref[_VP:, :] = jnp.dot(emb, wt2_ref[...],
                              preferred_element_type=jnp.float32
                              )[:_VP].astype(jnp.float8_e4m3fn)
    c_ref[...] = jnp.dot(wrbft_ref[...], wt3_ref[...],
                         preferred_element_type=jnp.float32
                         ).astype(jnp.bfloat16)


def _edge_body(zs_ref, zd_ref, rbft_ref, ab_ref, c_ref, m_ref):
    be = m_ref.shape[0]
    sub = jax.lax.broadcasted_iota(jnp.int32, (_VP, be), 0)
    zs = jnp.broadcast_to(zs_ref[...].reshape(1, be), (_VP, be))
    zd = jnp.broadcast_to(zd_ref[...].reshape(1, be), (_VP, be))
    oht = jnp.concatenate(
        [(zs == sub), (zd == sub)], axis=0).astype(jnp.float8_e4m3fn)
    m_ref[...] = (jax.lax.dot_general(
        oht, ab_ref[...], (((0,), (0,)), ((), ())),
        preferred_element_type=jnp.float32)
        + jax.lax.dot_general(
            rbft_ref[...], c_ref[...], (((0,), (0,)), ((), ())),
            preferred_element_type=jnp.float32))


def kernel(Z, edge_index, rbf, emb_table, W_rbf, b_rbf, W_dense, b_dense):
    N = Z.shape[0]
    E = edge_index.shape[1]
    EMB = emb_table.shape[1]
    NR = rbf.shape[1]

    Wd_t = W_dense.T
    Wt1 = Wd_t[:EMB]
    Wt2 = Wd_t[EMB:2 * EMB]
    Wt3 = Wd_t[2 * EMB:]
    Wrbf_t = W_rbf.T
    Zi = Z.astype(jnp.int32)
    Zc = Zi.reshape(N, 1)

    h, ab, C = pl.pallas_call(
        _tables_body,
        out_shape=(
            jax.ShapeDtypeStruct((N, EMB), jnp.float32),
            jax.ShapeDtypeStruct((2 * _VP, EMB), jnp.float8_e4m3fn),
            jax.ShapeDtypeStruct((NR, EMB), jnp.bfloat16),
        ),
    )(Zc, emb_table, Wt1, Wt2, Wt3, Wrbf_t,
      b_rbf.reshape(1, EMB), b_dense.reshape(1, EMB))

    rbft = rbf.T.astype(jnp.bfloat16)


    idx2d = edge_index.reshape(1, 2 * E).astype(jnp.int32)

    mesh = plsc.VectorSubcoreMesh(core_axis_name="c", subcore_axis_name="s")
    W = 1024

    @pl.kernel(out_type=jax.ShapeDtypeStruct((2 * E // W, W), jnp.int32),
               mesh=mesh,
               compiler_params=_SC_PARAMS,
               scratch_types=[pltpu.VMEM((N,), jnp.int32),
                              pltpu.SemaphoreType.DMA])
    def _classids(z_hbm, i_hbm, o_hbm, z_vmem, sem):
        pltpu.async_copy(z_hbm, z_vmem, sem).wait()

        def body(i_vmem, o_vmem):
            irow = i_vmem.at[0]
            orow = o_vmem.at[0]

            @pl.loop(0, W, step=16)
            def _(k):
                vidx = irow[pl.ds(k, 16)]
                orow[pl.ds(k, 16)] = plsc.load_gather(z_vmem, [vidx])

        pltpu.emit_pipeline(
            body,
            grid=(2 * E // W,),
            in_specs=[pl.BlockSpec((1, W), lambda i: (0, i))],
            out_specs=[pl.BlockSpec((1, W), lambda i: (i, 0))],
            core_axis_name=("c", "s"),
            dimension_semantics=(pltpu.PARALLEL,),
        )(i_hbm, o_hbm)

    zsd = _classids(Zi, idx2d).reshape(2, 1, E)

    BE = 32000
    NB = E // BE
    m = pl.pallas_call(
        _edge_body,
        grid=(NB,),
        in_specs=[
            pl.BlockSpec((1, 1, BE), lambda i: (0, 0, i)),
            pl.BlockSpec((1, 1, BE), lambda i: (1, 0, i)),
            pl.BlockSpec((NR, BE), lambda i: (0, i)),
            pl.BlockSpec((2 * _VP, EMB), lambda i: (0, 0)),
            pl.BlockSpec((NR, EMB), lambda i: (0, 0)),
        ],
        out_specs=pl.BlockSpec((BE, EMB), lambda i: (i, 0)),
        out_shape=jax.ShapeDtypeStruct((E, EMB), jnp.float32),
    )(zsd, zsd, rbft, ab, C)

    return (h, m)


# speedup vs baseline: 1.1045x; 1.0026x over previous
import dataclasses

import jax
import jax.numpy as jnp
from jax.experimental import pallas as pl
from jax.experimental.pallas import tpu as pltpu
from jax.experimental.pallas import tpu_sc as plsc

_SC_PARAMS = pltpu.CompilerParams()
if "needs_layout_passes" in pltpu.CompilerParams.__dataclass_fields__:
    _SC_PARAMS = dataclasses.replace(_SC_PARAMS, needs_layout_passes=False)


_VP = 104


def _tables_body(z_ref, emb_ref, wt1_ref, wt2_ref, wt3_ref, wrbft_ref,
                 brbf_ref, bdense_ref, h_ref, ab_ref, c_ref):
    n = z_ref.shape[0]
    v = emb_ref.shape[0]
    emb = emb_ref[...]
    if v < 128:
        emb = jnp.concatenate(
            [emb, jnp.zeros((128 - v, emb.shape[1]), emb.dtype)], axis=0)
    lane = jax.lax.broadcasted_iota(jnp.int32, (n, 128), 1)
    oh = (z_ref[...] == lane).astype(jnp.float32)
    h_ref[...] = jnp.dot(oh, emb, preferred_element_type=jnp.float32)
    bias = jnp.dot(brbf_ref[...], wt3_ref[...],
                   preferred_element_type=jnp.float32) + bdense_ref[...]
    ab_ref[:_VP, :] = (jnp.dot(emb, wt1_ref[...],
                               preferred_element_type=jnp.float32)
                       + bias)[:_VP].astype(jnp.float8_e4m3fn)
    ab_ref[_VP:, :] = jnp.dot(emb, wt2_ref[...],
                              preferred_element_type=jnp.float32
                              )[:_VP].astype(jnp.float8_e4m3fn)
    c_ref[...] = jnp.dot(wrbft_ref[...], wt3_ref[...],
                         preferred_element_type=jnp.float32
                         ).astype(jnp.bfloat16)


def _edge_body(zs_ref, zd_ref, rbft_ref, ab_ref, c_ref, m_ref):
    be = m_ref.shape[0]
    sub = jax.lax.broadcasted_iota(jnp.int32, (_VP, be), 0)
    zs = jnp.broadcast_to(zs_ref[...].reshape(1, be), (_VP, be))
    zd = jnp.broadcast_to(zd_ref[...].reshape(1, be), (_VP, be))
    oht = jnp.concatenate(
        [(zs == sub), (zd == sub)], axis=0).astype(jnp.float8_e4m3fn)
    m_ref[...] = (jax.lax.dot_general(
        oht, ab_ref[...], (((0,), (0,)), ((), ())),
        preferred_element_type=jnp.float32)
        + jax.lax.dot_general(
            rbft_ref[...], c_ref[...], (((0,), (0,)), ((), ())),
            preferred_element_type=jnp.float32))


def kernel(Z, edge_index, rbf, emb_table, W_rbf, b_rbf, W_dense, b_dense):
    N = Z.shape[0]
    E = edge_index.shape[1]
    EMB = emb_table.shape[1]
    NR = rbf.shape[1]

    Wd_t = W_dense.T
    Wt1 = Wd_t[:EMB]
    Wt2 = Wd_t[EMB:2 * EMB]
    Wt3 = Wd_t[2 * EMB:]
    Wrbf_t = W_rbf.T
    Zi = Z.astype(jnp.int32)
    Zc = Zi.reshape(N, 1)

    h, ab, C = pl.pallas_call(
        _tables_body,
        out_shape=(
            jax.ShapeDtypeStruct((N, EMB), jnp.float32),
            jax.ShapeDtypeStruct((2 * _VP, EMB), jnp.float8_e4m3fn),
            jax.ShapeDtypeStruct((NR, EMB), jnp.bfloat16),
        ),
    )(Zc, emb_table, Wt1, Wt2, Wt3, Wrbf_t,
      b_rbf.reshape(1, EMB), b_dense.reshape(1, EMB))

    rbft = rbf.T.astype(jnp.bfloat16)


    idx2d = edge_index.reshape(1, 2 * E).astype(jnp.int32)

    mesh = plsc.VectorSubcoreMesh(core_axis_name="c", subcore_axis_name="s")
    W = 2560

    @pl.kernel(out_type=jax.ShapeDtypeStruct((2 * E // W, W), jnp.int32),
               mesh=mesh,
               compiler_params=_SC_PARAMS,
               scratch_types=[pltpu.VMEM((N,), jnp.int32),
                              pltpu.SemaphoreType.DMA])
    def _classids(z_hbm, i_hbm, o_hbm, z_vmem, sem):
        pltpu.async_copy(z_hbm, z_vmem, sem).wait()

        def body(i_vmem, o_vmem):
            irow = i_vmem.at[0]
            orow = o_vmem.at[0]

            @pl.loop(0, W, step=16)
            def _(k):
                vidx = irow[pl.ds(k, 16)]
                orow[pl.ds(k, 16)] = plsc.load_gather(z_vmem, [vidx])

        pltpu.emit_pipeline(
            body,
            grid=(2 * E // W,),
            in_specs=[pl.BlockSpec((1, W), lambda i: (0, i))],
            out_specs=[pl.BlockSpec((1, W), lambda i: (i, 0))],
            core_axis_name=("c", "s"),
            dimension_semantics=(pltpu.PARALLEL,),
        )(i_hbm, o_hbm)

    zsd = _classids(Zi, idx2d).reshape(2, 1, E)

    BE = 32000
    NB = E // BE
    m = pl.pallas_call(
        _edge_body,
        grid=(NB,),
        in_specs=[
            pl.BlockSpec((1, 1, BE), lambda i: (0, 0, i)),
            pl.BlockSpec((1, 1, BE), lambda i: (1, 0, i)),
            pl.BlockSpec((NR, BE), lambda i: (0, i)),
            pl.BlockSpec((2 * _VP, EMB), lambda i: (0, 0)),
            pl.BlockSpec((NR, EMB), lambda i: (0, 0)),
        ],
        out_specs=pl.BlockSpec((BE, EMB), lambda i: (i, 0)),
        out_shape=jax.ShapeDtypeStruct((E, EMB), jnp.float32),
    )(zsd, zsd, rbft, ab, C)

    return (h, m)
